# baseline (speedup 1.0000x reference)
import functools

import jax
import jax.numpy as jnp
from jax import lax
from jax.experimental import pallas as pl
from jax.experimental.pallas import tpu as pltpu
from jax.experimental.pallas import tpu_sc as plsc

_SDR = 2048
_CONT = 384
_COLS = 2432
_ROWS_OUT = 4672
_TILE = 16
_NT = _ROWS_OUT // _TILE
_NC = 2
_NS = 16
_NW = _NC * _NS
_KMAX = (_NT + _NW - 1) // _NW


def _tile_offset(t):
    return jnp.where(t < 4, 4032,
                     jnp.where(t < 32, 3520,
                               jnp.where(t < 36, 3008, -576)))


def _tile_copies(t, sdrs, conts, zeros, out, sem):
    r0 = t * _TILE
    src = r0 + _tile_offset(t)
    data = [
        pltpu.make_async_copy(sdrs.at[pl.ds(src, _TILE)],
                              out.at[pl.ds(r0, _TILE), pl.ds(0, _SDR)], sem),
        pltpu.make_async_copy(conts.at[pl.ds(src, _TILE)],
                              out.at[pl.ds(r0, _TILE), pl.ds(_SDR, _CONT)], sem),
    ]
    last = [
        pltpu.make_async_copy(sdrs.at[pl.ds(3520, 1)],
                              out.at[pl.ds(4096, 1), pl.ds(0, _SDR)], sem),
        pltpu.make_async_copy(conts.at[pl.ds(3520, 1)],
                              out.at[pl.ds(4096, 1), pl.ds(_SDR, _CONT)], sem),
    ]
    zero = pltpu.make_async_copy(zeros, out.at[pl.ds(r0, _TILE)], sem)
    return data, last, zero


def _sc_body(sdrs, conts, zeros, out, sem):
    wid = lax.axis_index("s") * _NC + lax.axis_index("c")

    for k in range(_KMAX):
        t = wid + _NW * k
        data, last, zero = _tile_copies(t, sdrs, conts, zeros, out, sem)

        @pl.when(t <= 255)
        def _():
            for c in data:
                c.start()

        @pl.when(t == 256)
        def _():
            zero.start()
            zero.wait()
            for c in last:
                c.start()

        @pl.when((t >= 257) & (t < _NT))
        def _():
            zero.start()

    for k in range(_KMAX):
        t = wid + _NW * k
        data, last, zero = _tile_copies(t, sdrs, conts, zeros, out, sem)

        @pl.when(t <= 255)
        def _():
            for c in data:
                c.wait()

        @pl.when(t == 256)
        def _():
            for c in last:
                c.wait()

        @pl.when((t >= 257) & (t < _NT))
        def _():
            zero.wait()


def kernel(sdrs, contents):
    zeros = jnp.zeros((_TILE, _COLS), jnp.float32)
    mesh = plsc.VectorSubcoreMesh(core_axis_name="c", subcore_axis_name="s")
    run = functools.partial(
        pl.kernel,
        mesh=mesh,
        out_type=jax.ShapeDtypeStruct((_ROWS_OUT, _COLS), jnp.float32),
        scratch_types=[pltpu.SemaphoreType.DMA],
    )(_sc_body)
    return run(sdrs, contents, zeros)


# speedup vs baseline: 25.3397x; 25.3397x over previous
import functools

import jax
import jax.numpy as jnp
from jax import lax
from jax.experimental import pallas as pl
from jax.experimental.pallas import tpu as pltpu
from jax.experimental.pallas import tpu_sc as plsc

_SDR = 2048
_CONT = 384
_COLS = 2432
_ROWS_OUT = 4672
_TILE = 16
_NT = _ROWS_OUT // _TILE
_NC = 2
_NS = 16
_NW = _NC * _NS
_NDATA = 8


def _tile_offset(t):
    return jnp.where(t < 4, 4032,
                     jnp.where(t < 32, 3520,
                               jnp.where(t < 36, 3008, -576)))


def _sc_body(sdrs, conts, zeros, out, sbuf, cbuf, zbuf,
             gs0, gs1, ws0, ws1, zs):
    wid = lax.axis_index("s") * _NC + lax.axis_index("c")
    gsems = (gs0, gs1)
    wsems = (ws0, ws1)

    def gdesc(k, s):
        t = wid + _NW * k
        src = t * _TILE + _tile_offset(t)
        return [
            pltpu.make_async_copy(sdrs.at[pl.ds(src, _TILE)],
                                  sbuf.at[s], gsems[s]),
            pltpu.make_async_copy(conts.at[pl.ds(src, _TILE)],
                                  cbuf.at[s], gsems[s]),
        ]

    def wdesc(k, s):
        r0 = (wid + _NW * k) * _TILE
        return [
            pltpu.make_async_copy(sbuf.at[s],
                                  out.at[pl.ds(r0, _TILE), pl.ds(0, _SDR)],
                                  wsems[s]),
            pltpu.make_async_copy(cbuf.at[s],
                                  out.at[pl.ds(r0, _TILE), pl.ds(_SDR, _CONT)],
                                  wsems[s]),
        ]

    zcopy = pltpu.make_async_copy(zeros, zbuf, zs)
    zcopy.start()

    for c in gdesc(0, 0):
        c.start()
    for c in gdesc(1, 1):
        c.start()
    for k in range(_NDATA):
        s = k & 1
        for c in gdesc(k, s):
            c.wait()
        for c in wdesc(k, s):
            c.start()
        for c in wdesc(k, s):
            c.wait()
        if k + 2 < _NDATA:
            for c in gdesc(k + 2, s):
                c.start()

    zcopy.wait()
    t8 = wid + _NW * _NDATA

    @pl.when(t8 == 256)
    def _():
        pltpu.sync_copy(zbuf, out.at[pl.ds(4096, _TILE)])
        pltpu.sync_copy(sdrs.at[pl.ds(3520, 1)], sbuf.at[0, pl.ds(0, 1)])
        pltpu.sync_copy(conts.at[pl.ds(3520, 1)], cbuf.at[0, pl.ds(0, 1)])
        pltpu.sync_copy(sbuf.at[0, pl.ds(0, 1)],
                        out.at[pl.ds(4096, 1), pl.ds(0, _SDR)])
        pltpu.sync_copy(cbuf.at[0, pl.ds(0, 1)],
                        out.at[pl.ds(4096, 1), pl.ds(_SDR, _CONT)])

    @pl.when(t8 >= 257)
    def _():
        pltpu.sync_copy(zbuf, out.at[pl.ds(t8 * _TILE, _TILE)])

    t9 = wid + _NW * (_NDATA + 1)

    @pl.when(t9 < _NT)
    def _():
        pltpu.sync_copy(zbuf, out.at[pl.ds(t9 * _TILE, _TILE)])


def kernel(sdrs, contents):
    zeros = jnp.zeros((_TILE, _COLS), jnp.float32)
    mesh = plsc.VectorSubcoreMesh(core_axis_name="c", subcore_axis_name="s")
    run = functools.partial(
        pl.kernel,
        mesh=mesh,
        out_type=jax.ShapeDtypeStruct((_ROWS_OUT, _COLS), jnp.float32),
        scratch_types=[
            pltpu.VMEM((2, _TILE, _SDR), jnp.float32),
            pltpu.VMEM((2, _TILE, _CONT), jnp.float32),
            pltpu.VMEM((_TILE, _COLS), jnp.float32),
            pltpu.SemaphoreType.DMA,
            pltpu.SemaphoreType.DMA,
            pltpu.SemaphoreType.DMA,
            pltpu.SemaphoreType.DMA,
            pltpu.SemaphoreType.DMA,
        ],
    )(_sc_body)
    return run(sdrs, contents, zeros)
